# baseline (speedup 1.0000x reference)
import jax
import jax.numpy as jnp
from jax.experimental import pallas as pl

EPS = 1e-10


def _norm_body(g_ref, out_ref):
    blk = g_ref[...]
    deg = jnp.sum(blk, axis=-1, keepdims=True)
    out_ref[...] = blk / (deg + EPS)


def _pallas_normalize(un_g):
    B, K, _ = un_g.shape
    R = 256
    return pl.pallas_call(
        _norm_body,
        grid=(B, K // R),
        in_specs=[pl.BlockSpec((1, R, K), lambda b, r: (b, r, 0))],
        out_specs=pl.BlockSpec((1, R, K), lambda b, r: (b, r, 0)),
        out_shape=jax.ShapeDtypeStruct((B, K, K), jnp.float32),
    )(un_g)


def kernel(g, h, W, b):
    weights = jnp.squeeze(h @ W + b, -1)
    scores = jax.nn.sigmoid(weights)
    num_nodes = g.shape[-1]
    k_nodes = max(2, int(0.5 * num_nodes))
    values, idx = jax.lax.top_k(scores, k_nodes)
    new_h = jnp.take_along_axis(h, idx[:, :, None], axis=1)
    new_h = new_h * values[:, :, None]
    un_g = jnp.take_along_axis(g, idx[:, :, None], axis=1)
    un_g = jnp.take_along_axis(un_g, idx[:, None, :], axis=2)
    g_new = _pallas_normalize(un_g)
    return (g_new, new_h, idx)
